# Initial kernel scaffold; baseline (speedup 1.0000x reference)
import jax
import jax.numpy as jnp
from jax.experimental import pallas as pl


def kernel(pred_score, target):
    raise NotImplementedError("write your pallas kernel here")


# speedup vs baseline: 3.1157x; 3.1157x over previous
import functools

import jax
import jax.numpy as jnp
from jax import lax
from jax.experimental import pallas as pl
from jax.experimental.pallas import tpu as pltpu

_N = 16384
_C = 768
_R = 1024
_GRID = _N // _R

_INF_BITS = 0x7F800000


def _loss_body(pred_ref, tgt_ref, loss_ref):
    x = pred_ref[...]
    t = tgt_ref[...]
    m = jnp.max(x, axis=1, keepdims=True)
    e = jnp.exp(x - m)
    s = jnp.sum(e, axis=1, keepdims=True)
    lse = m + jnp.log(s)
    col = lax.broadcasted_iota(jnp.int32, (_R, _C), 1)
    xt = jnp.sum(jnp.where(col == t, x, 0.0), axis=1, keepdims=True)
    loss_ref[...] = lse - xt


def _combine_body(loss_ref, tgt_ref, total_ref, ner_ref, noner_ref):
    loss = loss_ref[...]
    t = tgt_ref[...]
    ner = t > 0
    ner_cnt = jnp.sum(jnp.where(ner, 1.0, 0.0))
    ner_sum = jnp.sum(jnp.where(ner, loss, 0.0))
    ner_loss = ner_sum / (ner_cnt + 1e-8)
    k = jnp.maximum(ner_cnt.astype(jnp.int32) // 2, 1)
    kf = k.astype(jnp.float32)

    v = jnp.where(t == 0, loss, 0.0)
    npos = jnp.sum(jnp.where(v > 0.0, 1.0, 0.0))
    nsteps = jnp.where(npos <= kf, 0, 31)

    def body(_, lohi):
        lo, hi = lohi
        mid = lo + ((hi - lo + 1) >> 1)
        tv = lax.bitcast_convert_type(mid, jnp.float32)
        cnt = jnp.sum(jnp.where(v >= tv, 1.0, 0.0))
        ge_k = cnt >= kf
        return (jnp.where(ge_k, mid, lo), jnp.where(ge_k, hi, mid - 1))

    lo, _ = lax.fori_loop(0, nsteps, body, (jnp.int32(0), jnp.int32(_INF_BITS)))
    tv = lax.bitcast_convert_type(lo, jnp.float32)
    gt = v > tv
    cnt_gt = jnp.sum(jnp.where(gt, 1.0, 0.0))
    sum_gt = jnp.sum(jnp.where(gt, v, 0.0))
    noner_loss = (sum_gt + (kf - cnt_gt) * tv) / kf

    ner_ref[0, 0] = ner_loss
    noner_ref[0, 0] = noner_loss
    total_ref[0, 0] = ner_loss * 3.0 + noner_loss * 0.3


@jax.jit
def _run(pred2d, tgt_col, tgt2d):
    loss = pl.pallas_call(
        _loss_body,
        grid=(_GRID,),
        in_specs=[
            pl.BlockSpec((_R, _C), lambda i: (i, 0)),
            pl.BlockSpec((_R, 1), lambda i: (i, 0)),
        ],
        out_specs=pl.BlockSpec((_R, 1), lambda i: (i, 0)),
        out_shape=jax.ShapeDtypeStruct((_N, 1), jnp.float32),
        compiler_params=pltpu.CompilerParams(
            dimension_semantics=("arbitrary",),
        ),
    )(pred2d, tgt_col)

    scalar = jax.ShapeDtypeStruct((1, 1), jnp.float32)
    smem = pl.BlockSpec(memory_space=pltpu.SMEM)
    total, ner_loss, noner_loss = pl.pallas_call(
        _combine_body,
        in_specs=[
            pl.BlockSpec((128, 128), lambda: (0, 0)),
            pl.BlockSpec((128, 128), lambda: (0, 0)),
        ],
        out_specs=[smem, smem, smem],
        out_shape=[scalar, scalar, scalar],
    )(loss.reshape(128, 128), tgt2d)
    return total[0, 0], ner_loss[0, 0], noner_loss[0, 0]


def kernel(pred_score, target):
    pred2d = pred_score.reshape(_N, _C)
    tgt = target.reshape(_N)
    return _run(pred2d, tgt.reshape(_N, 1), tgt.reshape(128, 128))


# speedup vs baseline: 3.3362x; 1.0707x over previous
import functools

import jax
import jax.numpy as jnp
from jax import lax
from jax.experimental import pallas as pl
from jax.experimental.pallas import tpu as pltpu

_N = 16384
_C = 768
_R = 2048
_GRID = _N // _R

_INF_BITS = 0x7F800000


def _loss_body(pred_ref, tgt_ref, loss_ref):
    x = pred_ref[...]
    t = tgt_ref[...]
    m = jnp.max(x, axis=1, keepdims=True)
    e = jnp.exp(x - m)
    s = jnp.sum(e, axis=1, keepdims=True)
    lse = m + jnp.log(s)
    col = lax.broadcasted_iota(jnp.int32, (_R, _C), 1)
    xt = jnp.sum(jnp.where(col == t, x, 0.0), axis=1, keepdims=True)
    loss_ref[...] = lse - xt


def _combine_body(loss_ref, tgt_ref, total_ref, ner_ref, noner_ref):
    loss = loss_ref[...]
    t = tgt_ref[...]
    ner = t > 0
    ner_cnt = jnp.sum(jnp.where(ner, 1.0, 0.0))
    ner_sum = jnp.sum(jnp.where(ner, loss, 0.0))
    ner_loss = ner_sum / (ner_cnt + 1e-8)
    k = jnp.maximum(ner_cnt.astype(jnp.int32) // 2, 1)
    kf = k.astype(jnp.float32)

    v = jnp.where(t == 0, loss, 0.0)
    npos = jnp.sum(jnp.where(v > 0.0, 1.0, 0.0))
    nsteps = jnp.where(npos <= kf, 0, 31)

    def body(_, lohi):
        lo, hi = lohi
        mid = lo + ((hi - lo + 1) >> 1)
        tv = lax.bitcast_convert_type(mid, jnp.float32)
        cnt = jnp.sum(jnp.where(v >= tv, 1.0, 0.0))
        ge_k = cnt >= kf
        return (jnp.where(ge_k, mid, lo), jnp.where(ge_k, hi, mid - 1))

    lo, _ = lax.fori_loop(0, nsteps, body, (jnp.int32(0), jnp.int32(_INF_BITS)))
    tv = lax.bitcast_convert_type(lo, jnp.float32)
    gt = v > tv
    cnt_gt = jnp.sum(jnp.where(gt, 1.0, 0.0))
    sum_gt = jnp.sum(jnp.where(gt, v, 0.0))
    noner_loss = (sum_gt + (kf - cnt_gt) * tv) / kf

    ner_ref[0, 0] = ner_loss
    noner_ref[0, 0] = noner_loss
    total_ref[0, 0] = ner_loss * 3.0 + noner_loss * 0.3


@jax.jit
def _run(pred2d, tgt_col, tgt2d):
    loss = pl.pallas_call(
        _loss_body,
        grid=(_GRID,),
        in_specs=[
            pl.BlockSpec((_R, _C), lambda i: (i, 0)),
            pl.BlockSpec((_R, 1), lambda i: (i, 0)),
        ],
        out_specs=pl.BlockSpec((_R, 1), lambda i: (i, 0)),
        out_shape=jax.ShapeDtypeStruct((_N, 1), jnp.float32),
        compiler_params=pltpu.CompilerParams(
            dimension_semantics=("arbitrary",),
        ),
    )(pred2d, tgt_col)

    scalar = jax.ShapeDtypeStruct((1, 1), jnp.float32)
    smem = pl.BlockSpec(memory_space=pltpu.SMEM)
    total, ner_loss, noner_loss = pl.pallas_call(
        _combine_body,
        in_specs=[
            pl.BlockSpec((128, 128), lambda: (0, 0)),
            pl.BlockSpec((128, 128), lambda: (0, 0)),
        ],
        out_specs=[smem, smem, smem],
        out_shape=[scalar, scalar, scalar],
    )(loss.reshape(128, 128), tgt2d)
    return total[0, 0], ner_loss[0, 0], noner_loss[0, 0]


def kernel(pred_score, target):
    pred2d = pred_score.reshape(_N, _C)
    tgt = target.reshape(_N)
    return _run(pred2d, tgt.reshape(_N, 1), tgt.reshape(128, 128))
